# Initial kernel scaffold; baseline (speedup 1.0000x reference)
import jax
import jax.numpy as jnp
from jax.experimental import pallas as pl


def kernel(node_feat, edge_index, labels, W0, b0, W1, b1, W2, b2, W3, b3, conv1_w, conv1_b, conv2_w, conv2_b, mlp_w1, mlp_b1, mlp_w2, mlp_b2):
    raise NotImplementedError("write your pallas kernel here")


# speedup vs baseline: 20.7266x; 20.7266x over previous
import functools

import jax
import jax.numpy as jnp
from jax import lax
from jax.experimental import pallas as pl
from jax.experimental.pallas import tpu as pltpu
from jax.experimental.pallas import tpu_sc as plsc

N = 100000
E = 3200000
B = 50
NPG = N // B
K = 6
NUM_CLASS = 12
HID = 128

LANES = 16
NSUB = 16
NCORE = 2
ER = E // 128
CH = 8
NT0 = 6256
ERH = 12504


def _sc_scatter_body(half_tables, table, src, dst, zeros, out,
                     sidx, didx, rows, acc, gsem):
    c = lax.axis_index("c")
    s = lax.axis_index("s")

    lo = pl.multiple_of(s * NT0, 8)
    ntl = N - (NSUB - 1) * NT0

    @pl.when(s < NSUB - 1)
    def _():
        pltpu.sync_copy(zeros.at[pl.ds(lo, NT0)], acc.at[pl.ds(lo, NT0)])

    @pl.when(s == NSUB - 1)
    def _():
        pltpu.sync_copy(zeros.at[pl.ds(lo, ntl)], acc.at[pl.ds(lo, ntl)])

    plsc.subcore_barrier()

    if half_tables == 2:
        n_chunks = ER // CH
        row_base = 0
        tab_off = c * N
        n_iters = (ER // CH + NSUB - 1) // NSUB
    else:
        n_chunks = ERH // CH - c
        row_base = c * ERH
        tab_off = None
        n_iters = (ERH // CH + NSUB - 1) // NSUB

    def step(k, _):
        chunk = k * NSUB + s

        @pl.when(chunk < n_chunks)
        def _():
            row0 = pl.multiple_of(row_base + chunk * CH, 8)
            pltpu.sync_copy(src.at[pl.ds(row0, CH)], sidx)
            pltpu.sync_copy(dst.at[pl.ds(row0, CH)], didx)
            if tab_off is not None:
                for j in range(CH):
                    for l in range(128 // LANES):
                        sl = pl.ds(l * LANES, LANES)
                        sidx[j, sl] = sidx[j, sl] + tab_off
            cps = [
                pltpu.async_copy(table.at[sidx.at[j]],
                                 rows.at[pl.ds(j * 128, 128)], gsem)
                for j in range(CH)
            ]
            for cp in cps:
                cp.wait()
            for j in range(CH):
                pltpu.sync_copy(rows.at[pl.ds(j * 128, 128)],
                                acc.at[didx.at[j]], add=True)

        return 0

    lax.fori_loop(0, n_iters, step, 0)

    plsc.subcore_barrier()

    @pl.when(s < NSUB - 1)
    def _():
        pltpu.sync_copy(acc.at[pl.ds(lo, NT0)], out.at[c, pl.ds(lo, NT0)])

    @pl.when(s == NSUB - 1)
    def _():
        pltpu.sync_copy(acc.at[pl.ds(lo, ntl)], out.at[c, pl.ds(lo, ntl)])


def _sc_scatter(half_tables, table, src, dst, zeros):
    mesh = plsc.VectorSubcoreMesh(core_axis_name="c", subcore_axis_name="s")
    fn = pl.kernel(
        functools.partial(_sc_scatter_body, half_tables),
        out_type=jax.ShapeDtypeStruct((2, N, LANES), jnp.float32),
        mesh=mesh,
        compiler_params=pltpu.CompilerParams(use_tc_tiling_on_sc=False),
        scratch_types=[
            pltpu.VMEM((CH, 128), jnp.int32),
            pltpu.VMEM((CH, 128), jnp.int32),
            pltpu.VMEM((CH * 128, LANES), jnp.float32),
            pltpu.VMEM_SHARED((N, LANES), jnp.float32),
            pltpu.SemaphoreType.DMA,
        ],
    )
    return fn(table, src, dst, zeros)


_RB = 4000


def _tck_l0_body(p_ref, x_ref, w_ref, b_ref, out_ref, dinv_ref):
    pool = p_ref[0] + p_ref[1]
    deg = pool[:, 11:12] + 1.0
    dinv = 1.0 / deg
    n2n = pool + x_ref[...]
    lin = jnp.dot(n2n, w_ref[...], preferred_element_type=jnp.float32)
    cur = jnp.tanh((lin + b_ref[...]) * dinv)
    out_ref[0] = cur[:, :16]
    out_ref[1] = cur[:, 16:]
    dinv_ref[...] = dinv


def _tck_mid_body(p_ref, t_ref, dinv_ref, w_ref, b_ref, out_ref):
    pool = jnp.concatenate([p_ref[0], p_ref[1]], axis=1)
    cur = jnp.concatenate([t_ref[0], t_ref[1]], axis=1)
    lin = jnp.dot(pool + cur, w_ref[...], preferred_element_type=jnp.float32)
    nxt = jnp.tanh((lin + b_ref[...]) * dinv_ref[...])
    out_ref[0] = nxt[:, :16]
    out_ref[1] = nxt[:, 16:]


def _tck_l3pre_body(t_ref, w_ref, out_ref):
    cur3 = jnp.concatenate([t_ref[0], t_ref[1]], axis=1)
    out_ref[...] = jnp.dot(cur3, w_ref[...], preferred_element_type=jnp.float32)


def _head_body(t1_ref, t2_ref, t3_ref, p3_ref, y3_ref, dinv_ref, lab_ref,
               b3_ref, w1t_ref, b1_ref, w2t_ref, b2_ref,
               mw1_ref, mb1_ref, mw2_ref, mb2_ref,
               logp_ref, loss_ref):
    g = pl.program_id(0)

    p3sum = p3_ref[0] + p3_ref[1] + y3_ref[...]
    cur4 = jnp.tanh((p3sum[:, 0:1] + b3_ref[0, 0]) * dinv_ref[...])

    cur1 = jnp.concatenate([t1_ref[0], t1_ref[1]], axis=1)
    cur2 = jnp.concatenate([t2_ref[0], t2_ref[1]], axis=1)
    cur3 = jnp.concatenate([t3_ref[0], t3_ref[1]], axis=1)

    iota = lax.broadcasted_iota(jnp.int32, (NPG, 1), 0)
    v = cur4
    rows1, rows2, rows3, rows4 = [], [], [], []
    for _ in range(K):
        m = jnp.max(v)
        idx = jnp.min(jnp.where(v == m, iota, jnp.int32(2147483647)))
        oh = (iota == idx).astype(jnp.float32)
        v = jnp.where(iota == idx, jnp.float32(-jnp.inf), v)
        rows1.append(jnp.sum(cur1 * oh, axis=0, keepdims=True))
        rows2.append(jnp.sum(cur2 * oh, axis=0, keepdims=True))
        rows3.append(jnp.sum(cur3 * oh, axis=0, keepdims=True))
        rows4.append(m.reshape(1, 1))
    pooled1 = jnp.concatenate(rows1, axis=0)
    pooled2 = jnp.concatenate(rows2, axis=0)
    pooled3 = jnp.concatenate(rows3, axis=0)
    pooled4 = jnp.concatenate(rows4, axis=0)

    w1t = w1t_ref[...]
    c1 = (jnp.dot(pooled1, w1t[0:32], preferred_element_type=jnp.float32)
          + jnp.dot(pooled2, w1t[32:64], preferred_element_type=jnp.float32)
          + jnp.dot(pooled3, w1t[64:96], preferred_element_type=jnp.float32)
          + pooled4 * w1t[96:97])
    c1 = jnp.maximum(c1 + b1_ref[...], 0.0)

    w2t = w2t_ref[...]
    c2 = jnp.zeros((1, 32), jnp.float32)
    for t in range(K // 2):
        mp = jnp.max(c1[2 * t:2 * t + 2, :], axis=0, keepdims=True)
        c2 = c2 + jnp.dot(mp, w2t[t * 16:(t + 1) * 16],
                          preferred_element_type=jnp.float32)
    c2 = jnp.maximum(c2 + b2_ref[...], 0.0)

    h = jnp.maximum(jnp.dot(c2, mw1_ref[...],
                            preferred_element_type=jnp.float32)
                    + mb1_ref[...], 0.0)
    logits = jnp.dot(h, mw2_ref[...],
                     preferred_element_type=jnp.float32) + mb2_ref[...]
    mx = jnp.max(logits, axis=1, keepdims=True)
    lse = mx + jnp.log(jnp.sum(jnp.exp(logits - mx), axis=1, keepdims=True))
    logp = logits - lse
    logp_ref[...] = logp[None]

    lab = lab_ref[0, 0, 0]
    iota12 = lax.broadcasted_iota(jnp.int32, (1, NUM_CLASS), 1)
    pick = jnp.sum(jnp.where(iota12 == lab, logp, 0.0))

    @pl.when(g == 0)
    def _():
        loss_ref[...] = jnp.zeros_like(loss_ref)

    loss_ref[...] += -pick / B


def _tc_layer0(p, x0p, w0p, b0):
    grid = N // _RB
    return pl.pallas_call(
        _tck_l0_body,
        grid=(grid,),
        in_specs=[
            pl.BlockSpec((2, _RB, 16), lambda i: (0, i, 0)),
            pl.BlockSpec((_RB, 16), lambda i: (i, 0)),
            pl.BlockSpec((16, 32), lambda i: (0, 0)),
            pl.BlockSpec((1, 32), lambda i: (0, 0)),
        ],
        out_specs=[
            pl.BlockSpec((2, _RB, 16), lambda i: (0, i, 0)),
            pl.BlockSpec((_RB, 1), lambda i: (i, 0)),
        ],
        out_shape=[
            jax.ShapeDtypeStruct((2, N, 16), jnp.float32),
            jax.ShapeDtypeStruct((N, 1), jnp.float32),
        ],
    )(p, x0p, w0p, b0)


def _tc_mid(p, t, dinv, w, b):
    grid = N // _RB
    return pl.pallas_call(
        _tck_mid_body,
        grid=(grid,),
        in_specs=[
            pl.BlockSpec((2, _RB, 16), lambda i: (0, i, 0)),
            pl.BlockSpec((2, _RB, 16), lambda i: (0, i, 0)),
            pl.BlockSpec((_RB, 1), lambda i: (i, 0)),
            pl.BlockSpec((32, 32), lambda i: (0, 0)),
            pl.BlockSpec((1, 32), lambda i: (0, 0)),
        ],
        out_specs=pl.BlockSpec((2, _RB, 16), lambda i: (0, i, 0)),
        out_shape=jax.ShapeDtypeStruct((2, N, 16), jnp.float32),
    )(p, t, dinv, w, b)


def _tc_l3pre(t, w3p):
    grid = N // _RB
    return pl.pallas_call(
        _tck_l3pre_body,
        grid=(grid,),
        in_specs=[
            pl.BlockSpec((2, _RB, 16), lambda i: (0, i, 0)),
            pl.BlockSpec((32, 16), lambda i: (0, 0)),
        ],
        out_specs=pl.BlockSpec((_RB, 16), lambda i: (i, 0)),
        out_shape=jax.ShapeDtypeStruct((N, 16), jnp.float32),
    )(t, w3p)


def _tc_head(t1, t2, t3, p3, y3p, dinv, labs, b3, w1t, b1, w2t, b2,
             mw1, mb1, mw2, mb2):
    full = lambda shape: pl.BlockSpec(shape, lambda g: tuple(0 for _ in shape))
    return pl.pallas_call(
        _head_body,
        grid=(B,),
        in_specs=[
            pl.BlockSpec((2, NPG, 16), lambda g: (0, g, 0)),
            pl.BlockSpec((2, NPG, 16), lambda g: (0, g, 0)),
            pl.BlockSpec((2, NPG, 16), lambda g: (0, g, 0)),
            pl.BlockSpec((2, NPG, 16), lambda g: (0, g, 0)),
            pl.BlockSpec((NPG, 16), lambda g: (g, 0)),
            pl.BlockSpec((NPG, 1), lambda g: (g, 0)),
            pl.BlockSpec((1, 1, 1), lambda g: (g, 0, 0)),
            full((1, 1)),
            full((97, 16)),
            full((1, 16)),
            full((48, 32)),
            full((1, 32)),
            full((32, HID)),
            full((1, HID)),
            full((HID, NUM_CLASS)),
            full((1, NUM_CLASS)),
        ],
        out_specs=[
            pl.BlockSpec((1, 1, NUM_CLASS), lambda g: (g, 0, 0)),
            pl.BlockSpec((1, 1), lambda g: (0, 0)),
        ],
        out_shape=[
            jax.ShapeDtypeStruct((B, 1, NUM_CLASS), jnp.float32),
            jax.ShapeDtypeStruct((1, 1), jnp.float32),
        ],
        compiler_params=pltpu.CompilerParams(
            dimension_semantics=("arbitrary",)),
    )(t1, t2, t3, p3, y3p, dinv, labs, b3, w1t, b1, w2t, b2,
      mw1, mb1, mw2, mb2)


def kernel(node_feat, edge_index, labels, W0, b0, W1, b1, W2, b2, W3, b3,
           conv1_w, conv1_b, conv2_w, conv2_b, mlp_w1, mlp_b1, mlp_w2, mlp_b2):
    f32 = jnp.float32
    src = edge_index[0].reshape(ER, 128)
    dst = edge_index[1].reshape(ER, 128)
    zeros = jnp.zeros((N, LANES), f32)

    x0p = jnp.concatenate(
        [node_feat, jnp.ones((N, 1), f32), jnp.zeros((N, 4), f32)], axis=1)
    w0p = jnp.concatenate([W0, jnp.zeros((5, 32), f32)], axis=0)
    w3p = jnp.concatenate([W3, jnp.zeros((32, 15), f32)], axis=1)

    p0 = _sc_scatter(1, x0p, src, dst, zeros)
    t1, dinv = _tc_layer0(p0, x0p, w0p, b0.reshape(1, 32))

    p1 = _sc_scatter(2, t1.reshape(2 * N, 16), src, dst, zeros)
    t2 = _tc_mid(p1, t1, dinv, W1, b1.reshape(1, 32))
    p2 = _sc_scatter(2, t2.reshape(2 * N, 16), src, dst, zeros)
    t3 = _tc_mid(p2, t2, dinv, W2, b2.reshape(1, 32))

    y3p = _tc_l3pre(t3, w3p)
    p3 = _sc_scatter(1, y3p, src, dst, zeros)

    w1t = conv1_w.T
    w2t = jnp.concatenate([conv2_w[:, :, t].T for t in range(3)], axis=0)
    logp, loss = _tc_head(
        t1, t2, t3, p3, y3p, dinv, labels.reshape(B, 1, 1).astype(jnp.int32),
        b3.reshape(1, 1), w1t, conv1_b.reshape(1, 16), w2t,
        conv2_b.reshape(1, 32), mlp_w1, mlp_b1.reshape(1, HID),
        mlp_w2, mlp_b2.reshape(1, NUM_CLASS))
    return logp.reshape(B, NUM_CLASS), loss.reshape(())


# speedup vs baseline: 27.2289x; 1.3137x over previous
import functools

import jax
import jax.numpy as jnp
from jax import lax
from jax.experimental import pallas as pl
from jax.experimental.pallas import tpu as pltpu
from jax.experimental.pallas import tpu_sc as plsc

N = 100000
E = 3200000
B = 50
NPG = N // B
K = 6
NUM_CLASS = 12
HID = 128

LANES = 16
NSUB = 16
NCORE = 2
ER = E // 128
CH = 4
NT0 = 6256
ERH = 12504


def _sc_scatter_body(half_tables, table, src, dst, zeros, out,
                     sidx0, didx0, rows0, sidx1, didx1, rows1, acc,
                     isem0, gsem0, ssem0, isem1, gsem1, ssem1):
    c = lax.axis_index("c")
    s = lax.axis_index("s")

    lo = pl.multiple_of(s * NT0, 8)
    ntl = N - (NSUB - 1) * NT0

    @pl.when(s < NSUB - 1)
    def _():
        pltpu.sync_copy(zeros.at[pl.ds(lo, NT0)], acc.at[pl.ds(lo, NT0)])

    @pl.when(s == NSUB - 1)
    def _():
        pltpu.sync_copy(zeros.at[pl.ds(lo, ntl)], acc.at[pl.ds(lo, ntl)])

    plsc.subcore_barrier()

    if half_tables == 2:
        n_chunks = ER // CH
        row_base = 0
        tab_off = c * N
        n_iters = (ER // CH + NSUB - 1) // NSUB
    else:
        n_chunks = ERH // CH - c * ((2 * ERH - ER) // CH)
        row_base = c * ERH
        tab_off = None
        n_iters = (ERH // CH + NSUB - 1) // NSUB

    bufs = ((sidx0, didx0, rows0, isem0, gsem0, ssem0),
            (sidx1, didx1, rows1, isem1, gsem1, ssem1))

    def chunk_row(i):
        return pl.multiple_of(row_base + (i * NSUB + s) * CH, CH)

    def fire_idx(i, b):
        si, di, _, isem, _, _ = bufs[b]
        r0 = chunk_row(i)
        pltpu.async_copy(src.at[pl.ds(r0, CH)], si, isem)
        pltpu.async_copy(dst.at[pl.ds(r0, CH)], di, isem)

    def wait_idx(i, b):
        si, di, _, isem, _, _ = bufs[b]
        r0 = chunk_row(i)
        pltpu.make_async_copy(src.at[pl.ds(r0, CH)], si, isem).wait()
        pltpu.make_async_copy(dst.at[pl.ds(r0, CH)], di, isem).wait()

    def fire_gather(b):
        si, _, rw, _, gsem, _ = bufs[b]
        if tab_off is not None:
            for j in range(CH):
                for l in range(128 // LANES):
                    sl = pl.ds(l * LANES, LANES)
                    si[j, sl] = si[j, sl] + tab_off
        for j in range(CH):
            pltpu.async_copy(table.at[si.at[j]],
                             rw.at[pl.ds(j * 128, 128)], gsem)

    def wait_gather(b):
        si, _, rw, _, gsem, _ = bufs[b]
        for j in range(CH):
            pltpu.make_async_copy(table.at[si.at[j]],
                                  rw.at[pl.ds(j * 128, 128)], gsem).wait()

    def fire_scatter(b):
        _, di, rw, _, _, ssem = bufs[b]
        for j in range(CH):
            pltpu.async_copy(rw.at[pl.ds(j * 128, 128)],
                             acc.at[di.at[j]], ssem, add=True)

    def drain_scatter(b):
        _, di, rw, _, _, ssem = bufs[b]
        for j in range(CH):
            pltpu.make_async_copy(rw.at[pl.ds(j * 128, 128)],
                                  acc.at[di.at[j]], ssem).wait()

    @pl.when(s < n_chunks)
    def _():
        fire_idx(0, 0)

    def step2(k2, _):
        for par in range(2):
            i = 2 * k2 + par
            cur = i * NSUB + s

            @pl.when(cur < n_chunks)
            def _():
                wait_idx(i, par)
                fire_gather(par)

            @pl.when((cur >= NSUB) & (cur - NSUB < n_chunks))
            def _():
                drain_scatter(1 - par)

            @pl.when(cur + NSUB < n_chunks)
            def _():
                fire_idx(i + 1, 1 - par)

            @pl.when(cur < n_chunks)
            def _():
                wait_gather(par)
                fire_scatter(par)

        return 0

    lax.fori_loop(0, n_iters // 2 + 1, step2, 0)

    plsc.subcore_barrier()

    @pl.when(s < NSUB - 1)
    def _():
        pltpu.sync_copy(acc.at[pl.ds(lo, NT0)], out.at[c, pl.ds(lo, NT0)])

    @pl.when(s == NSUB - 1)
    def _():
        pltpu.sync_copy(acc.at[pl.ds(lo, ntl)], out.at[c, pl.ds(lo, ntl)])


def _sc_scatter(half_tables, table, src, dst, zeros):
    mesh = plsc.VectorSubcoreMesh(core_axis_name="c", subcore_axis_name="s")
    fn = pl.kernel(
        functools.partial(_sc_scatter_body, half_tables),
        out_type=jax.ShapeDtypeStruct((2, N, LANES), jnp.float32),
        mesh=mesh,
        compiler_params=pltpu.CompilerParams(use_tc_tiling_on_sc=False),
        scratch_types=[
            pltpu.VMEM((CH, 128), jnp.int32),
            pltpu.VMEM((CH, 128), jnp.int32),
            pltpu.VMEM((CH * 128, LANES), jnp.float32),
            pltpu.VMEM((CH, 128), jnp.int32),
            pltpu.VMEM((CH, 128), jnp.int32),
            pltpu.VMEM((CH * 128, LANES), jnp.float32),
            pltpu.VMEM_SHARED((N, LANES), jnp.float32),
            pltpu.SemaphoreType.DMA,
            pltpu.SemaphoreType.DMA,
            pltpu.SemaphoreType.DMA,
            pltpu.SemaphoreType.DMA,
            pltpu.SemaphoreType.DMA,
            pltpu.SemaphoreType.DMA,
        ],
    )
    return fn(table, src, dst, zeros)


_RB = 4000


def _tck_l0_body(p_ref, x_ref, w_ref, b_ref, out_ref, dinv_ref):
    pool = p_ref[0] + p_ref[1]
    deg = pool[:, 11:12] + 1.0
    dinv = 1.0 / deg
    n2n = pool + x_ref[...]
    lin = jnp.dot(n2n, w_ref[...], preferred_element_type=jnp.float32)
    cur = jnp.tanh((lin + b_ref[...]) * dinv)
    out_ref[0] = cur[:, :16]
    out_ref[1] = cur[:, 16:]
    dinv_ref[...] = dinv


def _tck_mid_body(p_ref, t_ref, dinv_ref, w_ref, b_ref, out_ref):
    pool = jnp.concatenate([p_ref[0], p_ref[1]], axis=1)
    cur = jnp.concatenate([t_ref[0], t_ref[1]], axis=1)
    lin = jnp.dot(pool + cur, w_ref[...], preferred_element_type=jnp.float32)
    nxt = jnp.tanh((lin + b_ref[...]) * dinv_ref[...])
    out_ref[0] = nxt[:, :16]
    out_ref[1] = nxt[:, 16:]


def _tck_l3pre_body(t_ref, w_ref, out_ref):
    cur3 = jnp.concatenate([t_ref[0], t_ref[1]], axis=1)
    out_ref[...] = jnp.dot(cur3, w_ref[...], preferred_element_type=jnp.float32)


def _head_body(t1_ref, t2_ref, t3_ref, p3_ref, y3_ref, dinv_ref, lab_ref,
               b3_ref, w1t_ref, b1_ref, w2t_ref, b2_ref,
               mw1_ref, mb1_ref, mw2_ref, mb2_ref,
               logp_ref, loss_ref):
    g = pl.program_id(0)

    p3sum = p3_ref[0] + p3_ref[1] + y3_ref[...]
    cur4 = jnp.tanh((p3sum[:, 0:1] + b3_ref[0, 0]) * dinv_ref[...])

    cur1 = jnp.concatenate([t1_ref[0], t1_ref[1]], axis=1)
    cur2 = jnp.concatenate([t2_ref[0], t2_ref[1]], axis=1)
    cur3 = jnp.concatenate([t3_ref[0], t3_ref[1]], axis=1)

    iota = lax.broadcasted_iota(jnp.int32, (NPG, 1), 0)
    v = cur4
    rows1, rows2, rows3, rows4 = [], [], [], []
    for _ in range(K):
        m = jnp.max(v)
        idx = jnp.min(jnp.where(v == m, iota, jnp.int32(2147483647)))
        oh = (iota == idx).astype(jnp.float32)
        v = jnp.where(iota == idx, jnp.float32(-jnp.inf), v)
        rows1.append(jnp.sum(cur1 * oh, axis=0, keepdims=True))
        rows2.append(jnp.sum(cur2 * oh, axis=0, keepdims=True))
        rows3.append(jnp.sum(cur3 * oh, axis=0, keepdims=True))
        rows4.append(m.reshape(1, 1))
    pooled1 = jnp.concatenate(rows1, axis=0)
    pooled2 = jnp.concatenate(rows2, axis=0)
    pooled3 = jnp.concatenate(rows3, axis=0)
    pooled4 = jnp.concatenate(rows4, axis=0)

    w1t = w1t_ref[...]
    c1 = (jnp.dot(pooled1, w1t[0:32], preferred_element_type=jnp.float32)
          + jnp.dot(pooled2, w1t[32:64], preferred_element_type=jnp.float32)
          + jnp.dot(pooled3, w1t[64:96], preferred_element_type=jnp.float32)
          + pooled4 * w1t[96:97])
    c1 = jnp.maximum(c1 + b1_ref[...], 0.0)

    w2t = w2t_ref[...]
    c2 = jnp.zeros((1, 32), jnp.float32)
    for t in range(K // 2):
        mp = jnp.max(c1[2 * t:2 * t + 2, :], axis=0, keepdims=True)
        c2 = c2 + jnp.dot(mp, w2t[t * 16:(t + 1) * 16],
                          preferred_element_type=jnp.float32)
    c2 = jnp.maximum(c2 + b2_ref[...], 0.0)

    h = jnp.maximum(jnp.dot(c2, mw1_ref[...],
                            preferred_element_type=jnp.float32)
                    + mb1_ref[...], 0.0)
    logits = jnp.dot(h, mw2_ref[...],
                     preferred_element_type=jnp.float32) + mb2_ref[...]
    mx = jnp.max(logits, axis=1, keepdims=True)
    lse = mx + jnp.log(jnp.sum(jnp.exp(logits - mx), axis=1, keepdims=True))
    logp = logits - lse
    logp_ref[...] = logp[None]

    lab = lab_ref[0, 0, 0]
    iota12 = lax.broadcasted_iota(jnp.int32, (1, NUM_CLASS), 1)
    pick = jnp.sum(jnp.where(iota12 == lab, logp, 0.0))

    @pl.when(g == 0)
    def _():
        loss_ref[...] = jnp.zeros_like(loss_ref)

    loss_ref[...] += -pick / B


def _tc_layer0(p, x0p, w0p, b0):
    grid = N // _RB
    return pl.pallas_call(
        _tck_l0_body,
        grid=(grid,),
        in_specs=[
            pl.BlockSpec((2, _RB, 16), lambda i: (0, i, 0)),
            pl.BlockSpec((_RB, 16), lambda i: (i, 0)),
            pl.BlockSpec((16, 32), lambda i: (0, 0)),
            pl.BlockSpec((1, 32), lambda i: (0, 0)),
        ],
        out_specs=[
            pl.BlockSpec((2, _RB, 16), lambda i: (0, i, 0)),
            pl.BlockSpec((_RB, 1), lambda i: (i, 0)),
        ],
        out_shape=[
            jax.ShapeDtypeStruct((2, N, 16), jnp.float32),
            jax.ShapeDtypeStruct((N, 1), jnp.float32),
        ],
    )(p, x0p, w0p, b0)


def _tc_mid(p, t, dinv, w, b):
    grid = N // _RB
    return pl.pallas_call(
        _tck_mid_body,
        grid=(grid,),
        in_specs=[
            pl.BlockSpec((2, _RB, 16), lambda i: (0, i, 0)),
            pl.BlockSpec((2, _RB, 16), lambda i: (0, i, 0)),
            pl.BlockSpec((_RB, 1), lambda i: (i, 0)),
            pl.BlockSpec((32, 32), lambda i: (0, 0)),
            pl.BlockSpec((1, 32), lambda i: (0, 0)),
        ],
        out_specs=pl.BlockSpec((2, _RB, 16), lambda i: (0, i, 0)),
        out_shape=jax.ShapeDtypeStruct((2, N, 16), jnp.float32),
    )(p, t, dinv, w, b)


def _tc_l3pre(t, w3p):
    grid = N // _RB
    return pl.pallas_call(
        _tck_l3pre_body,
        grid=(grid,),
        in_specs=[
            pl.BlockSpec((2, _RB, 16), lambda i: (0, i, 0)),
            pl.BlockSpec((32, 16), lambda i: (0, 0)),
        ],
        out_specs=pl.BlockSpec((_RB, 16), lambda i: (i, 0)),
        out_shape=jax.ShapeDtypeStruct((N, 16), jnp.float32),
    )(t, w3p)


def _tc_head(t1, t2, t3, p3, y3p, dinv, labs, b3, w1t, b1, w2t, b2,
             mw1, mb1, mw2, mb2):
    full = lambda shape: pl.BlockSpec(shape, lambda g: tuple(0 for _ in shape))
    return pl.pallas_call(
        _head_body,
        grid=(B,),
        in_specs=[
            pl.BlockSpec((2, NPG, 16), lambda g: (0, g, 0)),
            pl.BlockSpec((2, NPG, 16), lambda g: (0, g, 0)),
            pl.BlockSpec((2, NPG, 16), lambda g: (0, g, 0)),
            pl.BlockSpec((2, NPG, 16), lambda g: (0, g, 0)),
            pl.BlockSpec((NPG, 16), lambda g: (g, 0)),
            pl.BlockSpec((NPG, 1), lambda g: (g, 0)),
            pl.BlockSpec((1, 1, 1), lambda g: (g, 0, 0)),
            full((1, 1)),
            full((97, 16)),
            full((1, 16)),
            full((48, 32)),
            full((1, 32)),
            full((32, HID)),
            full((1, HID)),
            full((HID, NUM_CLASS)),
            full((1, NUM_CLASS)),
        ],
        out_specs=[
            pl.BlockSpec((1, 1, NUM_CLASS), lambda g: (g, 0, 0)),
            pl.BlockSpec((1, 1), lambda g: (0, 0)),
        ],
        out_shape=[
            jax.ShapeDtypeStruct((B, 1, NUM_CLASS), jnp.float32),
            jax.ShapeDtypeStruct((1, 1), jnp.float32),
        ],
        compiler_params=pltpu.CompilerParams(
            dimension_semantics=("arbitrary",)),
    )(t1, t2, t3, p3, y3p, dinv, labs, b3, w1t, b1, w2t, b2,
      mw1, mb1, mw2, mb2)


def kernel(node_feat, edge_index, labels, W0, b0, W1, b1, W2, b2, W3, b3,
           conv1_w, conv1_b, conv2_w, conv2_b, mlp_w1, mlp_b1, mlp_w2, mlp_b2):
    f32 = jnp.float32
    src = edge_index[0].reshape(ER, 128)
    dst = edge_index[1].reshape(ER, 128)
    zeros = jnp.zeros((N, LANES), f32)

    x0p = jnp.concatenate(
        [node_feat, jnp.ones((N, 1), f32), jnp.zeros((N, 4), f32)], axis=1)
    w0p = jnp.concatenate([W0, jnp.zeros((5, 32), f32)], axis=0)
    w3p = jnp.concatenate([W3, jnp.zeros((32, 15), f32)], axis=1)

    p0 = _sc_scatter(1, x0p, src, dst, zeros)
    t1, dinv = _tc_layer0(p0, x0p, w0p, b0.reshape(1, 32))

    p1 = _sc_scatter(2, t1.reshape(2 * N, 16), src, dst, zeros)
    t2 = _tc_mid(p1, t1, dinv, W1, b1.reshape(1, 32))
    p2 = _sc_scatter(2, t2.reshape(2 * N, 16), src, dst, zeros)
    t3 = _tc_mid(p2, t2, dinv, W2, b2.reshape(1, 32))

    y3p = _tc_l3pre(t3, w3p)
    p3 = _sc_scatter(1, y3p, src, dst, zeros)

    w1t = conv1_w.T
    w2t = jnp.concatenate([conv2_w[:, :, t].T for t in range(3)], axis=0)
    logp, loss = _tc_head(
        t1, t2, t3, p3, y3p, dinv, labels.reshape(B, 1, 1).astype(jnp.int32),
        b3.reshape(1, 1), w1t, conv1_b.reshape(1, 16), w2t,
        conv2_b.reshape(1, 32), mlp_w1, mlp_b1.reshape(1, HID),
        mlp_w2, mlp_b2.reshape(1, NUM_CLASS))
    return logp.reshape(B, NUM_CLASS), loss.reshape(())


# speedup vs baseline: 28.5633x; 1.0490x over previous
import functools

import jax
import jax.numpy as jnp
from jax import lax
from jax.experimental import pallas as pl
from jax.experimental.pallas import tpu as pltpu
from jax.experimental.pallas import tpu_sc as plsc

N = 100000
E = 3200000
B = 50
NPG = N // B
K = 6
NUM_CLASS = 12
HID = 128

LANES = 16
NSUB = 16
NCORE = 2
ER = E // 128
CH = 4
NT0 = 6256
ERH = 12504


def _sc_scatter_body(half_tables, table, src, dst, zeros, out,
                     sidx0, didx0, rows0, sidx1, didx1, rows1, acc,
                     isem0, gsem0, ssem0, isem1, gsem1, ssem1):
    c = lax.axis_index("c")
    s = lax.axis_index("s")

    lo = pl.multiple_of(s * NT0, 8)
    ntl = N - (NSUB - 1) * NT0

    @pl.when(s < NSUB - 1)
    def _():
        pltpu.sync_copy(zeros.at[pl.ds(lo, NT0)], acc.at[pl.ds(lo, NT0)])

    @pl.when(s == NSUB - 1)
    def _():
        pltpu.sync_copy(zeros.at[pl.ds(lo, ntl)], acc.at[pl.ds(lo, ntl)])

    plsc.subcore_barrier()

    if half_tables == 2:
        n_chunks = ER // CH
        row_base = 0
        tab_off = c * N
        n_iters = (ER // CH + NSUB - 1) // NSUB
    else:
        n_chunks = ERH // CH - c * ((2 * ERH - ER) // CH)
        row_base = c * ERH
        tab_off = None
        n_iters = (ERH // CH + NSUB - 1) // NSUB

    bufs = ((sidx0, didx0, rows0, isem0, gsem0, ssem0),
            (sidx1, didx1, rows1, isem1, gsem1, ssem1))

    def chunk_row(i):
        return pl.multiple_of(row_base + (i * NSUB + s) * CH, CH)

    def fire_idx(i, b):
        si, di, _, isem, _, _ = bufs[b]
        r0 = chunk_row(i)
        pltpu.async_copy(src.at[pl.ds(r0, CH)], si, isem)
        pltpu.async_copy(dst.at[pl.ds(r0, CH)], di, isem)

    def wait_idx(i, b):
        si, di, _, isem, _, _ = bufs[b]
        r0 = chunk_row(i)
        pltpu.make_async_copy(src.at[pl.ds(r0, CH)], si, isem).wait()
        pltpu.make_async_copy(dst.at[pl.ds(r0, CH)], di, isem).wait()

    def fire_gather(b):
        si, _, rw, _, gsem, _ = bufs[b]
        if tab_off is not None:
            for j in range(CH):
                for l in range(128 // LANES):
                    sl = pl.ds(l * LANES, LANES)
                    si[j, sl] = si[j, sl] + tab_off
        for j in range(CH):
            pltpu.async_copy(table.at[si.at[j]],
                             rw.at[pl.ds(j * 128, 128)], gsem)

    def wait_gather(b):
        si, _, rw, _, gsem, _ = bufs[b]
        for j in range(CH):
            pltpu.make_async_copy(table.at[si.at[j]],
                                  rw.at[pl.ds(j * 128, 128)], gsem).wait()

    def fire_scatter(b):
        _, di, rw, _, _, ssem = bufs[b]
        for j in range(CH):
            pltpu.async_copy(rw.at[pl.ds(j * 128, 128)],
                             acc.at[di.at[j]], ssem, add=True)

    def drain_scatter(b):
        _, di, rw, _, _, ssem = bufs[b]
        for j in range(CH):
            pltpu.make_async_copy(rw.at[pl.ds(j * 128, 128)],
                                  acc.at[di.at[j]], ssem).wait()

    @pl.when(s < n_chunks)
    def _():
        fire_idx(0, 0)

    def step2(k2, _):
        for par in range(2):
            i = 2 * k2 + par
            cur = i * NSUB + s

            @pl.when(cur < n_chunks)
            def _():
                wait_idx(i, par)
                fire_gather(par)

            @pl.when((cur >= NSUB) & (cur - NSUB < n_chunks))
            def _():
                drain_scatter(1 - par)

            @pl.when(cur + NSUB < n_chunks)
            def _():
                fire_idx(i + 1, 1 - par)

            @pl.when(cur < n_chunks)
            def _():
                wait_gather(par)
                fire_scatter(par)

        return 0

    lax.fori_loop(0, n_iters // 2 + 1, step2, 0)

    plsc.subcore_barrier()

    @pl.when(s < NSUB - 1)
    def _():
        pltpu.sync_copy(acc.at[pl.ds(lo, NT0)], out.at[c, pl.ds(lo, NT0)])

    @pl.when(s == NSUB - 1)
    def _():
        pltpu.sync_copy(acc.at[pl.ds(lo, ntl)], out.at[c, pl.ds(lo, ntl)])


def _sc_scatter(half_tables, table, src, dst, zeros):
    mesh = plsc.VectorSubcoreMesh(core_axis_name="c", subcore_axis_name="s")
    fn = pl.kernel(
        functools.partial(_sc_scatter_body, half_tables),
        out_type=jax.ShapeDtypeStruct((2, N, LANES), jnp.float32),
        mesh=mesh,
        compiler_params=pltpu.CompilerParams(use_tc_tiling_on_sc=False),
        scratch_types=[
            pltpu.VMEM((CH, 128), jnp.int32),
            pltpu.VMEM((CH, 128), jnp.int32),
            pltpu.VMEM((CH * 128, LANES), jnp.float32),
            pltpu.VMEM((CH, 128), jnp.int32),
            pltpu.VMEM((CH, 128), jnp.int32),
            pltpu.VMEM((CH * 128, LANES), jnp.float32),
            pltpu.VMEM_SHARED((N, LANES), jnp.float32),
            pltpu.SemaphoreType.DMA,
            pltpu.SemaphoreType.DMA,
            pltpu.SemaphoreType.DMA,
            pltpu.SemaphoreType.DMA,
            pltpu.SemaphoreType.DMA,
            pltpu.SemaphoreType.DMA,
        ],
    )
    return fn(table, src, dst, zeros)


_RB = 4000


def _tck_l0_body(p_ref, x_ref, w_ref, b_ref, out_ref, dinv_ref):
    pool = p_ref[0] + p_ref[1]
    deg = pool[:, 11:12] + 1.0
    dinv = 1.0 / deg
    n2n = pool + x_ref[...]
    lin = jnp.dot(n2n, w_ref[...], preferred_element_type=jnp.float32)
    cur = jnp.tanh((lin + b_ref[...]) * dinv)
    out_ref[0] = cur[:, :16]
    out_ref[1] = cur[:, 16:]
    dinv_ref[...] = dinv


def _tck_mid_body(p_ref, t_ref, dinv_ref, w_ref, b_ref, out_ref):
    pool = jnp.concatenate([p_ref[0], p_ref[1]], axis=1)
    cur = jnp.concatenate([t_ref[0], t_ref[1]], axis=1)
    lin = jnp.dot(pool + cur, w_ref[...], preferred_element_type=jnp.float32)
    nxt = jnp.tanh((lin + b_ref[...]) * dinv_ref[...])
    out_ref[0] = nxt[:, :16]
    out_ref[1] = nxt[:, 16:]


def _tck_l3pre_body(t_ref, w_ref, out_ref):
    cur3 = jnp.concatenate([t_ref[0], t_ref[1]], axis=1)
    out_ref[...] = jnp.dot(cur3, w_ref[...], preferred_element_type=jnp.float32)


def _tck_l3post_body(p3_ref, y3_ref, dinv_ref, b3_ref, out_ref):
    p3sum = p3_ref[0] + p3_ref[1] + y3_ref[...]
    out_ref[...] = jnp.tanh((p3sum[:, 0:1] + b3_ref[0, 0]) * dinv_ref[...])


def _head_body(t1_ref, t2_ref, t3_ref, v_ref, lab_ref,
               w1t_ref, b1_ref, w2t_ref, b2_ref,
               mw1_ref, mb1_ref, mw2_ref, mb2_ref,
               logp_ref, loss_ref):
    g = pl.program_id(0)

    cur1 = jnp.concatenate([t1_ref[0], t1_ref[1]], axis=1)
    cur2 = jnp.concatenate([t2_ref[0], t2_ref[1]], axis=1)
    cur3 = jnp.concatenate([t3_ref[0], t3_ref[1]], axis=1)

    iota = lax.broadcasted_iota(jnp.int32, (1, NPG), 1)
    v = v_ref[0]
    ohs, rows4 = [], []
    for _ in range(K):
        m = jnp.max(v)
        idx = jnp.min(jnp.where(v == m, iota, jnp.int32(2147483647)))
        oh = (iota == idx).astype(jnp.float32)
        v = jnp.where(iota == idx, jnp.float32(-jnp.inf), v)
        ohs.append(oh)
        rows4.append(m.reshape(1, 1))
    ohp = jnp.concatenate(ohs, axis=0)
    dot = lambda a, b: jax.lax.dot_general(
        a, b, (((1,), (0,)), ((), ())), preferred_element_type=jnp.float32)
    pooled1 = dot(ohp, cur1)
    pooled2 = dot(ohp, cur2)
    pooled3 = dot(ohp, cur3)
    pooled4 = jnp.concatenate(rows4, axis=0)

    w1t = w1t_ref[...]
    c1 = (jnp.dot(pooled1, w1t[0:32], preferred_element_type=jnp.float32)
          + jnp.dot(pooled2, w1t[32:64], preferred_element_type=jnp.float32)
          + jnp.dot(pooled3, w1t[64:96], preferred_element_type=jnp.float32)
          + pooled4 * w1t[96:97])
    c1 = jnp.maximum(c1 + b1_ref[...], 0.0)

    w2t = w2t_ref[...]
    c2 = jnp.zeros((1, 32), jnp.float32)
    for t in range(K // 2):
        mp = jnp.max(c1[2 * t:2 * t + 2, :], axis=0, keepdims=True)
        c2 = c2 + jnp.dot(mp, w2t[t * 16:(t + 1) * 16],
                          preferred_element_type=jnp.float32)
    c2 = jnp.maximum(c2 + b2_ref[...], 0.0)

    h = jnp.maximum(jnp.dot(c2, mw1_ref[...],
                            preferred_element_type=jnp.float32)
                    + mb1_ref[...], 0.0)
    logits = jnp.dot(h, mw2_ref[...],
                     preferred_element_type=jnp.float32) + mb2_ref[...]
    mx = jnp.max(logits, axis=1, keepdims=True)
    lse = mx + jnp.log(jnp.sum(jnp.exp(logits - mx), axis=1, keepdims=True))
    logp = logits - lse
    logp_ref[...] = logp[None]

    lab = lab_ref[0, 0, 0]
    iota12 = lax.broadcasted_iota(jnp.int32, (1, NUM_CLASS), 1)
    pick = jnp.sum(jnp.where(iota12 == lab, logp, 0.0))

    @pl.when(g == 0)
    def _():
        loss_ref[...] = jnp.zeros_like(loss_ref)

    loss_ref[...] += -pick / B


def _tc_layer0(p, x0p, w0p, b0):
    grid = N // _RB
    return pl.pallas_call(
        _tck_l0_body,
        grid=(grid,),
        in_specs=[
            pl.BlockSpec((2, _RB, 16), lambda i: (0, i, 0)),
            pl.BlockSpec((_RB, 16), lambda i: (i, 0)),
            pl.BlockSpec((16, 32), lambda i: (0, 0)),
            pl.BlockSpec((1, 32), lambda i: (0, 0)),
        ],
        out_specs=[
            pl.BlockSpec((2, _RB, 16), lambda i: (0, i, 0)),
            pl.BlockSpec((_RB, 1), lambda i: (i, 0)),
        ],
        out_shape=[
            jax.ShapeDtypeStruct((2, N, 16), jnp.float32),
            jax.ShapeDtypeStruct((N, 1), jnp.float32),
        ],
    )(p, x0p, w0p, b0)


def _tc_mid(p, t, dinv, w, b):
    grid = N // _RB
    return pl.pallas_call(
        _tck_mid_body,
        grid=(grid,),
        in_specs=[
            pl.BlockSpec((2, _RB, 16), lambda i: (0, i, 0)),
            pl.BlockSpec((2, _RB, 16), lambda i: (0, i, 0)),
            pl.BlockSpec((_RB, 1), lambda i: (i, 0)),
            pl.BlockSpec((32, 32), lambda i: (0, 0)),
            pl.BlockSpec((1, 32), lambda i: (0, 0)),
        ],
        out_specs=pl.BlockSpec((2, _RB, 16), lambda i: (0, i, 0)),
        out_shape=jax.ShapeDtypeStruct((2, N, 16), jnp.float32),
    )(p, t, dinv, w, b)


def _tc_l3pre(t, w3p):
    grid = N // _RB
    return pl.pallas_call(
        _tck_l3pre_body,
        grid=(grid,),
        in_specs=[
            pl.BlockSpec((2, _RB, 16), lambda i: (0, i, 0)),
            pl.BlockSpec((32, 16), lambda i: (0, 0)),
        ],
        out_specs=pl.BlockSpec((_RB, 16), lambda i: (i, 0)),
        out_shape=jax.ShapeDtypeStruct((N, 16), jnp.float32),
    )(t, w3p)


def _tc_l3post(p3, y3p, dinv, b3):
    grid = N // _RB
    return pl.pallas_call(
        _tck_l3post_body,
        grid=(grid,),
        in_specs=[
            pl.BlockSpec((2, _RB, 16), lambda i: (0, i, 0)),
            pl.BlockSpec((_RB, 16), lambda i: (i, 0)),
            pl.BlockSpec((_RB, 1), lambda i: (i, 0)),
            pl.BlockSpec((1, 1), lambda i: (0, 0)),
        ],
        out_specs=pl.BlockSpec((_RB, 1), lambda i: (i, 0)),
        out_shape=jax.ShapeDtypeStruct((N, 1), jnp.float32),
    )(p3, y3p, dinv, b3)


def _tc_head(t1, t2, t3, v2d, labs, w1t, b1, w2t, b2,
             mw1, mb1, mw2, mb2):
    full = lambda shape: pl.BlockSpec(shape, lambda g: tuple(0 for _ in shape))
    return pl.pallas_call(
        _head_body,
        grid=(B,),
        in_specs=[
            pl.BlockSpec((2, NPG, 16), lambda g: (0, g, 0)),
            pl.BlockSpec((2, NPG, 16), lambda g: (0, g, 0)),
            pl.BlockSpec((2, NPG, 16), lambda g: (0, g, 0)),
            pl.BlockSpec((1, 1, NPG), lambda g: (g, 0, 0)),
            pl.BlockSpec((1, 1, 1), lambda g: (g, 0, 0)),
            full((97, 16)),
            full((1, 16)),
            full((48, 32)),
            full((1, 32)),
            full((32, HID)),
            full((1, HID)),
            full((HID, NUM_CLASS)),
            full((1, NUM_CLASS)),
        ],
        out_specs=[
            pl.BlockSpec((1, 1, NUM_CLASS), lambda g: (g, 0, 0)),
            pl.BlockSpec((1, 1), lambda g: (0, 0)),
        ],
        out_shape=[
            jax.ShapeDtypeStruct((B, 1, NUM_CLASS), jnp.float32),
            jax.ShapeDtypeStruct((1, 1), jnp.float32),
        ],
        compiler_params=pltpu.CompilerParams(
            dimension_semantics=("arbitrary",)),
    )(t1, t2, t3, v2d, labs, w1t, b1, w2t, b2,
      mw1, mb1, mw2, mb2)


def kernel(node_feat, edge_index, labels, W0, b0, W1, b1, W2, b2, W3, b3,
           conv1_w, conv1_b, conv2_w, conv2_b, mlp_w1, mlp_b1, mlp_w2, mlp_b2):
    f32 = jnp.float32
    src = edge_index[0].reshape(ER, 128)
    dst = edge_index[1].reshape(ER, 128)
    zeros = jnp.zeros((N, LANES), f32)

    x0p = jnp.concatenate(
        [node_feat, jnp.ones((N, 1), f32), jnp.zeros((N, 4), f32)], axis=1)
    w0p = jnp.concatenate([W0, jnp.zeros((5, 32), f32)], axis=0)
    w3p = jnp.concatenate([W3, jnp.zeros((32, 15), f32)], axis=1)

    p0 = _sc_scatter(1, x0p, src, dst, zeros)
    t1, dinv = _tc_layer0(p0, x0p, w0p, b0.reshape(1, 32))

    p1 = _sc_scatter(2, t1.reshape(2 * N, 16), src, dst, zeros)
    t2 = _tc_mid(p1, t1, dinv, W1, b1.reshape(1, 32))
    p2 = _sc_scatter(2, t2.reshape(2 * N, 16), src, dst, zeros)
    t3 = _tc_mid(p2, t2, dinv, W2, b2.reshape(1, 32))

    y3p = _tc_l3pre(t3, w3p)
    p3 = _sc_scatter(1, y3p, src, dst, zeros)

    w1t = conv1_w.T
    w2t = jnp.concatenate([conv2_w[:, :, t].T for t in range(3)], axis=0)
    cur4 = _tc_l3post(p3, y3p, dinv, b3.reshape(1, 1))
    v2d = cur4.reshape(B, 1, NPG)
    logp, loss = _tc_head(
        t1, t2, t3, v2d, labels.reshape(B, 1, 1).astype(jnp.int32),
        w1t, conv1_b.reshape(1, 16), w2t,
        conv2_b.reshape(1, 32), mlp_w1, mlp_b1.reshape(1, HID),
        mlp_w2, mlp_b2.reshape(1, NUM_CLASS))
    return logp.reshape(B, NUM_CLASS), loss.reshape(())
